# baseline (speedup 1.0000x reference)
import functools

import jax
import jax.numpy as jnp
from jax import lax
from jax.experimental import pallas as pl
from jax.experimental.pallas import tpu as pltpu
from jax.experimental.pallas import tpu_sc as plsc

N = 10000
H = 128
B = 64
E = 320000
NC = 2
NS = 16
NW = NC * NS
CH = 128
NCHUNK = 80
EPT = NCHUNK * CH
EHALF = NS * EPT
HALFX = EHALF + CH
EPAD = NC * EHALF
TPR = 640
TPRA = 648
MDIV = 6554


def _tc_pos_body(dst_ref, pos_ref, ptab_ref):
    bins = lax.broadcasted_iota(jnp.int32, (NS, 1), 0)
    lane = lax.broadcasted_iota(jnp.int32, (NS, 128), 1)
    for c in range(NC):
        d = dst_ref[c]
        w = lax.shift_right_logical(d * MDIV, 22)
        oh = (w[None, :] == bins).astype(jnp.int32)
        incl = oh
        sh = 1
        while sh < EHALF:
            shifted = jnp.concatenate(
                [jnp.zeros((NS, sh), jnp.int32), incl[:, :EHALF - sh]], axis=1)
            incl = incl + shifted
            sh *= 2
        run = jnp.int32(0)
        starts = []
        for wb in range(NS):
            starts.append(run)
            run = run + incl[wb, EHALF - 1]
        prelv = jnp.stack(starts).reshape(NS, 1)
        pos_rel = jnp.sum(oh * (prelv + incl - 1), axis=0)
        pos_ref[c] = pos_rel + c * HALFX
        endv = jnp.concatenate(
            [prelv[1:], jnp.full((1, 1), EHALF, jnp.int32)], axis=0)
        ptab_ref[c] = jnp.where(lane == 0, prelv,
                                jnp.where(lane == 1, endv, 0))


_tc_pos = pl.pallas_call(
    _tc_pos_body,
    out_shape=(jax.ShapeDtypeStruct((NC, EHALF), jnp.int32),
               jax.ShapeDtypeStruct((NC, NS, 128), jnp.int32)),
)


def _sc_bin_body(src_hbm, dst_hbm, pos_hbm, bs_hbm, bd_hbm,
                 sv, dv, pv, zv, qsem):
    c = lax.axis_index("c")
    s = lax.axis_index("s")

    pltpu.sync_copy(src_hbm.at[c, s], sv)
    pltpu.sync_copy(dst_hbm.at[c, s], dv)
    pltpu.sync_copy(pos_hbm.at[c, s], pv)

    @pl.when(s == 0)
    def _zp():
        for i in range(CH // 16):
            zv[pl.ds(i * 16, 16)] = jnp.zeros((16,), jnp.int32)
        zoff = pl.multiple_of(c * HALFX + EHALF, 8)
        pltpu.sync_copy(zv, bs_hbm.at[pl.ds(zoff, CH)])
        pltpu.sync_copy(zv, bd_hbm.at[pl.ds(zoff, CH)])

    def _grp(g, _):
        for q in range(4):
            j = g * 4 + q
            pltpu.async_copy(sv.at[j], bs_hbm.at[pv.at[j]], qsem)
            pltpu.async_copy(dv.at[j], bd_hbm.at[pv.at[j]], qsem)
        for q in range(4):
            j = g * 4 + q
            pltpu.make_async_copy(sv.at[j], bs_hbm.at[pv.at[j]], qsem).wait()
            pltpu.make_async_copy(dv.at[j], bd_hbm.at[pv.at[j]], qsem).wait()
        return 0
    lax.fori_loop(0, NCHUNK // 4, _grp, 0)


@functools.lru_cache(maxsize=None)
def _make_sc_bin():
    mesh = plsc.VectorSubcoreMesh(core_axis_name="c", subcore_axis_name="s")
    return pl.kernel(
        _sc_bin_body,
        mesh=mesh,
        out_type=(jax.ShapeDtypeStruct((NC * HALFX,), jnp.int32),
                  jax.ShapeDtypeStruct((NC * HALFX,), jnp.int32)),
        scratch_types=(
            [pltpu.VMEM((NCHUNK, CH), jnp.int32)] * 3
            + [pltpu.VMEM((CH,), jnp.int32),
               pltpu.SemaphoreType.DMA]
        ),
    )


def _sc_agg_body(x_hbm, bs_hbm, bd_hbm, ptab_hbm, out_hbm,
                 pvv, sidx, didx, buf, acc, isem, gsem):
    c = lax.axis_index("c")
    s = lax.axis_index("s")
    lanes = lax.iota(jnp.int32, 16)

    pltpu.sync_copy(ptab_hbm.at[c, s], pvv)
    pv_lo = pvv[pl.ds(0, 16)]
    start = pv_lo[0]
    end = pv_lo[1]
    astart = lax.shift_left(lax.shift_right_logical(start, 3), 3)
    m = lax.shift_right_logical(end - astart + (CH - 1), 7)
    fbase = pl.multiple_of(c * HALFX + astart, 8)
    rowbase = s * TPR

    def _zr(r, _):
        for g in range(H // 16):
            acc[r, pl.ds(g * 16, 16)] = jnp.zeros((16,), jnp.float32)
        return 0
    lax.fori_loop(0, TPRA, _zr, 0)

    def _chunk(j, _):
        off = pl.multiple_of(fbase + j * CH, 8)
        pltpu.async_copy(bs_hbm.at[pl.ds(off, CH)], sidx, isem)
        pltpu.async_copy(bd_hbm.at[pl.ds(off, CH)], didx, isem)
        pltpu.make_async_copy(bs_hbm.at[pl.ds(off, CH)], sidx, isem).wait()
        pltpu.make_async_copy(bd_hbm.at[pl.ds(off, CH)], didx, isem).wait()
        pltpu.async_copy(x_hbm.at[sidx], buf, gsem)
        pltpu.make_async_copy(x_hbm.at[sidx], buf, gsem).wait()
        gl0 = astart + j * CH
        for q in range(CH // 16):
            dvec = didx[pl.ds(q * 16, 16)]
            gpos = gl0 + q * 16 + lanes
            ok = jnp.logical_and(gpos >= start, gpos < end)
            ldv = jnp.where(ok, dvec - rowbase, jnp.int32(TPR))
            for k in range(16):
                row = ldv[k]
                e = q * 16 + k
                for g in range(H // 16):
                    plsc.addupdate(acc.at[row, pl.ds(g * 16, 16)],
                                   buf[e, pl.ds(g * 16, 16)])
        return 0
    lax.fori_loop(0, m, _chunk, 0)

    @pl.when(s < NS - 1)
    def _cp():
        pltpu.sync_copy(acc.at[pl.ds(0, TPR)], out_hbm.at[c, pl.ds(s * TPR, TPR)])

    @pl.when(s == NS - 1)
    def _cpl():
        nlast = N - (NS - 1) * TPR
        pltpu.sync_copy(acc.at[pl.ds(0, nlast)],
                        out_hbm.at[c, pl.ds((NS - 1) * TPR, nlast)])


@functools.lru_cache(maxsize=None)
def _make_sc_agg():
    mesh = plsc.VectorSubcoreMesh(core_axis_name="c", subcore_axis_name="s")
    return pl.kernel(
        _sc_agg_body,
        mesh=mesh,
        out_type=jax.ShapeDtypeStruct((NC, N, H), jnp.float32),
        scratch_types=(
            [pltpu.VMEM((128,), jnp.int32)]
            + [pltpu.VMEM((CH,), jnp.int32)] * 2
            + [pltpu.VMEM((CH, H), jnp.float32)]
            + [pltpu.VMEM((TPRA, H), jnp.float32)]
            + [pltpu.SemaphoreType.DMA] * 2
        ),
    )


def _dense1_body(x_ref, agg_ref, Wa_ref, ba_ref, g_ref, be_ref, Wb_ref, bb_ref, out_ref):
    h = x_ref[...] + agg_ref[0] + agg_ref[1]
    h = jnp.dot(h, Wa_ref[...], preferred_element_type=jnp.float32) + ba_ref[...]
    m = jnp.mean(h, axis=0, keepdims=True)
    cc = h - m
    v = jnp.mean(cc * cc, axis=0, keepdims=True)
    h = g_ref[...] * cc * lax.rsqrt(v + 1e-5) + be_ref[...]
    h = jnp.maximum(h, 0.0)
    h = jnp.dot(h, Wb_ref[...], preferred_element_type=jnp.float32) + bb_ref[...]
    out_ref[...] = jnp.maximum(h, 0.0)


_dense1 = pl.pallas_call(
    _dense1_body,
    out_shape=jax.ShapeDtypeStruct((N, H), jnp.float32),
)


def _dense2_body(h1_ref, agg_ref, batch_ref, Wa_ref, ba_ref, g_ref, be_ref,
                 Wb_ref, bb_ref, Wl1_ref, bl1_ref, Wl2_ref, bl2_ref,
                 sig_ref, lin_ref):
    h1 = h1_ref[...]
    h = h1 + agg_ref[0] + agg_ref[1]
    h = jnp.dot(h, Wa_ref[...], preferred_element_type=jnp.float32) + ba_ref[...]
    m = jnp.mean(h, axis=0, keepdims=True)
    cc = h - m
    v = jnp.mean(cc * cc, axis=0, keepdims=True)
    h = g_ref[...] * cc * lax.rsqrt(v + 1e-5) + be_ref[...]
    h = jnp.maximum(h, 0.0)
    h = jnp.dot(h, Wb_ref[...], preferred_element_type=jnp.float32) + bb_ref[...]
    h2 = jnp.maximum(h, 0.0)

    bvec = batch_ref[...]
    seg = lax.broadcasted_iota(jnp.int32, (1, B), 1)
    onehot = (bvec == seg).astype(jnp.float32)
    dn = (((0,), (0,)), ((), ()))
    h1_sum = lax.dot_general(onehot, h1, dn, preferred_element_type=jnp.float32)
    h2_sum = lax.dot_general(onehot, h2, dn, preferred_element_type=jnp.float32)

    neg = jnp.float32(-jnp.inf)
    rowid = lax.broadcasted_iota(jnp.int32, (B, 1), 0)

    def _seg_max(b, carry):
        m1acc, m2acc = carry
        mask = bvec == b
        m1 = jnp.max(jnp.where(mask, h1, neg), axis=0, keepdims=True)
        m2 = jnp.max(jnp.where(mask, h2, neg), axis=0, keepdims=True)
        rowsel = rowid == b
        return (jnp.where(rowsel, m1, m1acc), jnp.where(rowsel, m2, m2acc))

    h1_max, h2_max = lax.fori_loop(
        0, B, _seg_max,
        (jnp.full((B, H), neg), jnp.full((B, H), neg)))

    hp = jnp.concatenate((h1_sum, h2_sum, h1_max, h2_max), axis=1)
    hh = jnp.dot(hp, Wl1_ref[...], preferred_element_type=jnp.float32) + bl1_ref[...]
    hh = jnp.maximum(hh, 0.0)
    hh = jnp.dot(hh, Wl2_ref[...], preferred_element_type=jnp.float32) + bl2_ref[...]
    lin_ref[...] = hh
    sig_ref[...] = jax.nn.sigmoid(hh)


_dense2 = pl.pallas_call(
    _dense2_body,
    out_shape=(jax.ShapeDtypeStruct((B, 1), jnp.float32),
               jax.ShapeDtypeStruct((B, 1), jnp.float32)),
)


def kernel(x, edge_index, batch, W1a, b1a, g1, be1, W1b, b1b, W2a, b2a, g2, be2,
           W2b, b2b, Wl1, bl1, Wl2, bl2):
    src = edge_index[0]
    dst = edge_index[1]
    pad = EPAD - E
    src4 = jnp.concatenate([src, jnp.zeros((pad,), jnp.int32)]).reshape(NC, NS, NCHUNK, CH)
    dst_p = jnp.concatenate([dst, jnp.full((pad,), N, jnp.int32)])
    dst4 = dst_p.reshape(NC, NS, NCHUNK, CH)

    pos, ptab = _tc_pos(dst_p.reshape(NC, EHALF))
    pos4 = pos.reshape(NC, NS, NCHUNK, CH)
    bs, bd = _make_sc_bin()(src4, dst4, pos4)
    _sc_agg = _make_sc_agg()
    agg1 = _sc_agg(x, bs, bd, ptab)
    h1 = _dense1(x, agg1, W1a, b1a.reshape(1, H), g1.reshape(1, H),
                 be1.reshape(1, H), W1b, b1b.reshape(1, H))
    agg2 = _sc_agg(h1, bs, bd, ptab)
    return _dense2(h1, agg2, batch.reshape(N, 1), W2a, b2a.reshape(1, H),
                   g2.reshape(1, H), be2.reshape(1, H), W2b, b2b.reshape(1, H),
                   Wl1, bl1.reshape(1, 4 * H), Wl2, bl2.reshape(1, 1))


# speedup vs baseline: 1.9561x; 1.9561x over previous
import functools

import jax
import jax.numpy as jnp
from jax import lax
from jax.experimental import pallas as pl
from jax.experimental.pallas import tpu as pltpu
from jax.experimental.pallas import tpu_sc as plsc

N = 10000
H = 128
B = 64
E = 320000
NC = 2
NS = 16
NW = NC * NS
CH = 128
NCHUNK = 80
EPT = NCHUNK * CH
EHALF = NS * EPT
HALFX = EHALF + CH
EPAD = NC * EHALF
TPR = 640
TPRA = 648
MDIV = 6554


def _tc_pos_body(dst_ref, pos_ref, ptab_ref):
    bins = lax.broadcasted_iota(jnp.int32, (NS, 1), 0)
    lane = lax.broadcasted_iota(jnp.int32, (NS, 128), 1)
    for c in range(NC):
        d = dst_ref[c]
        w = lax.shift_right_logical(d * MDIV, 22)
        oh = (w[None, :] == bins).astype(jnp.int32)
        incl = oh
        sh = 1
        while sh < EHALF:
            shifted = jnp.concatenate(
                [jnp.zeros((NS, sh), jnp.int32), incl[:, :EHALF - sh]], axis=1)
            incl = incl + shifted
            sh *= 2
        run = jnp.int32(0)
        starts = []
        for wb in range(NS):
            starts.append(run)
            run = run + incl[wb, EHALF - 1]
        prelv = jnp.stack(starts).reshape(NS, 1)
        pos_rel = jnp.sum(oh * (prelv + incl - 1), axis=0)
        pos_ref[c] = pos_rel
        endv = jnp.concatenate(
            [prelv[1:], jnp.full((1, 1), EHALF, jnp.int32)], axis=0)
        ptab_ref[c] = jnp.where(lane == 0, prelv,
                                jnp.where(lane == 1, endv, 0))


_tc_pos = pl.pallas_call(
    _tc_pos_body,
    out_shape=(jax.ShapeDtypeStruct((NC, EHALF), jnp.int32),
               jax.ShapeDtypeStruct((NC, NS, 128), jnp.int32)),
)


CPB = HALFX // NS


def _sc_bin_body(src_hbm, dst_hbm, pos_hbm, bs_hbm, bd_hbm,
                 sv, dv, pv, zv, bounce, bs_sp, bd_sp, qsem):
    c = lax.axis_index("c")
    s = lax.axis_index("s")

    pltpu.sync_copy(src_hbm.at[c, s], sv)
    pltpu.sync_copy(dst_hbm.at[c, s], dv)
    pltpu.sync_copy(pos_hbm.at[c, s], pv)

    @pl.when(s == 0)
    def _zp():
        for i in range(CH // 16):
            zv[pl.ds(i * 16, 16)] = jnp.zeros((16,), jnp.int32)
        pltpu.sync_copy(zv, bs_sp.at[pl.ds(EHALF, CH)])
        pltpu.sync_copy(zv, bd_sp.at[pl.ds(EHALF, CH)])

    def _grp(g, _):
        for q in range(4):
            j = g * 4 + q
            pltpu.async_copy(sv.at[j], bs_sp.at[pv.at[j]], qsem)
            pltpu.async_copy(dv.at[j], bd_sp.at[pv.at[j]], qsem)
        for q in range(4):
            j = g * 4 + q
            pltpu.make_async_copy(sv.at[j], bs_sp.at[pv.at[j]], qsem).wait()
            pltpu.make_async_copy(dv.at[j], bd_sp.at[pv.at[j]], qsem).wait()
        return 0
    lax.fori_loop(0, NCHUNK // 4, _grp, 0)
    plsc.subcore_barrier()

    hoff = pl.multiple_of(c * HALFX + s * CPB, 8)
    pltpu.sync_copy(bs_sp.at[pl.ds(s * CPB, CPB)], bounce)
    pltpu.sync_copy(bounce, bs_hbm.at[pl.ds(hoff, CPB)])
    pltpu.sync_copy(bd_sp.at[pl.ds(s * CPB, CPB)], bounce)
    pltpu.sync_copy(bounce, bd_hbm.at[pl.ds(hoff, CPB)])


@functools.lru_cache(maxsize=None)
def _make_sc_bin():
    mesh = plsc.VectorSubcoreMesh(core_axis_name="c", subcore_axis_name="s")
    return pl.kernel(
        _sc_bin_body,
        mesh=mesh,
        out_type=(jax.ShapeDtypeStruct((NC * HALFX,), jnp.int32),
                  jax.ShapeDtypeStruct((NC * HALFX,), jnp.int32)),
        scratch_types=(
            [pltpu.VMEM((NCHUNK, CH), jnp.int32)] * 3
            + [pltpu.VMEM((CH,), jnp.int32),
               pltpu.VMEM((CPB,), jnp.int32),
               pltpu.VMEM_SHARED((HALFX,), jnp.int32),
               pltpu.VMEM_SHARED((HALFX,), jnp.int32),
               pltpu.SemaphoreType.DMA]
        ),
    )


def _sc_agg_body(x_hbm, bs_hbm, bd_hbm, ptab_hbm, out_hbm,
                 pvv, si0, si1, di0, di1, bu0, bu1, acc,
                 i0, i1, g0s, g1s):
    sidx = (si0, si1)
    didx = (di0, di1)
    bufs = (bu0, bu1)
    isem = (i0, i1)
    gsem = (g0s, g1s)
    c = lax.axis_index("c")
    s = lax.axis_index("s")
    lanes = lax.iota(jnp.int32, 16)

    pltpu.sync_copy(ptab_hbm.at[c, s], pvv)
    pv_lo = pvv[pl.ds(0, 16)]
    start = pv_lo[0]
    end = pv_lo[1]
    astart = lax.shift_left(lax.shift_right_logical(start, 3), 3)
    m = lax.shift_right_logical(end - astart + (CH - 1), 7)
    fbase = pl.multiple_of(c * HALFX + astart, 8)
    rowbase = s * TPR

    def _zr(r, _):
        for g in range(H // 16):
            acc[r, pl.ds(g * 16, 16)] = jnp.zeros((16,), jnp.float32)
        return 0
    lax.fori_loop(0, TPRA, _zr, 0)

    def _fetch_idx(j, b):
        off = pl.multiple_of(fbase + j * CH, 8)
        pltpu.async_copy(bs_hbm.at[pl.ds(off, CH)], sidx[b], isem[b])
        pltpu.async_copy(bd_hbm.at[pl.ds(off, CH)], didx[b], isem[b])

    def _wait_idx(j, b):
        off = pl.multiple_of(fbase + j * CH, 8)
        pltpu.make_async_copy(bs_hbm.at[pl.ds(off, CH)], sidx[b], isem[b]).wait()
        pltpu.make_async_copy(bd_hbm.at[pl.ds(off, CH)], didx[b], isem[b]).wait()

    def _accumulate(j, b):
        gl0 = astart + j * CH
        for q in range(CH // 16):
            dvec = didx[b][pl.ds(q * 16, 16)]
            gpos = gl0 + q * 16 + lanes
            ok = jnp.logical_and(gpos >= start, gpos < end)
            ldv = jnp.where(ok, dvec - rowbase, jnp.int32(TPR))
            for k in range(16):
                row = ldv[k]
                e = q * 16 + k
                for g in range(H // 16):
                    plsc.addupdate(acc.at[row, pl.ds(g * 16, 16)],
                                   bufs[b][e, pl.ds(g * 16, 16)])

    for b in range(2):
        @pl.when(b < m)
        def _():
            _fetch_idx(b, b)
    @pl.when(0 < m)
    def _():
        _wait_idx(0, 0)
        pltpu.async_copy(x_hbm.at[sidx[0]], bufs[0], gsem[0])

    def _outer(t, _):
        for b in range(2):
            j = t * 2 + b
            bn = (b + 1) % 2

            @pl.when(j < m)
            def _():
                pltpu.make_async_copy(x_hbm.at[sidx[b]], bufs[b], gsem[b]).wait()

                @pl.when(j + 1 < m)
                def _():
                    _wait_idx(j + 1, bn)
                    pltpu.async_copy(x_hbm.at[sidx[bn]], bufs[bn], gsem[bn])
                _accumulate(j, b)

                @pl.when(j + 2 < m)
                def _():
                    _fetch_idx(j + 2, b)
        return 0
    lax.fori_loop(0, lax.shift_right_logical(m + 1, 1), _outer, 0)

    @pl.when(s < NS - 1)
    def _cp():
        pltpu.sync_copy(acc.at[pl.ds(0, TPR)], out_hbm.at[c, pl.ds(s * TPR, TPR)])

    @pl.when(s == NS - 1)
    def _cpl():
        nlast = N - (NS - 1) * TPR
        pltpu.sync_copy(acc.at[pl.ds(0, nlast)],
                        out_hbm.at[c, pl.ds((NS - 1) * TPR, nlast)])


@functools.lru_cache(maxsize=None)
def _make_sc_agg():
    mesh = plsc.VectorSubcoreMesh(core_axis_name="c", subcore_axis_name="s")
    return pl.kernel(
        _sc_agg_body,
        mesh=mesh,
        out_type=jax.ShapeDtypeStruct((NC, N, H), jnp.float32),
        scratch_types=(
            [pltpu.VMEM((128,), jnp.int32)]
            + [pltpu.VMEM((CH,), jnp.int32)] * 4
            + [pltpu.VMEM((CH, H), jnp.float32)] * 2
            + [pltpu.VMEM((TPRA, H), jnp.float32)]
            + [pltpu.SemaphoreType.DMA] * 4
        ),
    )


def _dense1_body(x_ref, agg_ref, Wa_ref, ba_ref, g_ref, be_ref, Wb_ref, bb_ref, out_ref):
    h = x_ref[...] + agg_ref[0] + agg_ref[1]
    h = jnp.dot(h, Wa_ref[...], preferred_element_type=jnp.float32) + ba_ref[...]
    m = jnp.mean(h, axis=0, keepdims=True)
    cc = h - m
    v = jnp.mean(cc * cc, axis=0, keepdims=True)
    h = g_ref[...] * cc * lax.rsqrt(v + 1e-5) + be_ref[...]
    h = jnp.maximum(h, 0.0)
    h = jnp.dot(h, Wb_ref[...], preferred_element_type=jnp.float32) + bb_ref[...]
    out_ref[...] = jnp.maximum(h, 0.0)


_dense1 = pl.pallas_call(
    _dense1_body,
    out_shape=jax.ShapeDtypeStruct((N, H), jnp.float32),
)


def _dense2_body(h1_ref, agg_ref, batch_ref, Wa_ref, ba_ref, g_ref, be_ref,
                 Wb_ref, bb_ref, Wl1_ref, bl1_ref, Wl2_ref, bl2_ref,
                 sig_ref, lin_ref):
    h1 = h1_ref[...]
    h = h1 + agg_ref[0] + agg_ref[1]
    h = jnp.dot(h, Wa_ref[...], preferred_element_type=jnp.float32) + ba_ref[...]
    m = jnp.mean(h, axis=0, keepdims=True)
    cc = h - m
    v = jnp.mean(cc * cc, axis=0, keepdims=True)
    h = g_ref[...] * cc * lax.rsqrt(v + 1e-5) + be_ref[...]
    h = jnp.maximum(h, 0.0)
    h = jnp.dot(h, Wb_ref[...], preferred_element_type=jnp.float32) + bb_ref[...]
    h2 = jnp.maximum(h, 0.0)

    bvec = batch_ref[...]
    seg = lax.broadcasted_iota(jnp.int32, (1, B), 1)
    onehot = (bvec == seg).astype(jnp.float32)
    dn = (((0,), (0,)), ((), ()))
    h1_sum = lax.dot_general(onehot, h1, dn, preferred_element_type=jnp.float32)
    h2_sum = lax.dot_general(onehot, h2, dn, preferred_element_type=jnp.float32)

    neg = jnp.float32(-jnp.inf)
    rowid = lax.broadcasted_iota(jnp.int32, (B, 1), 0)

    def _seg_max(b, carry):
        m1acc, m2acc = carry
        mask = bvec == b
        m1 = jnp.max(jnp.where(mask, h1, neg), axis=0, keepdims=True)
        m2 = jnp.max(jnp.where(mask, h2, neg), axis=0, keepdims=True)
        rowsel = rowid == b
        return (jnp.where(rowsel, m1, m1acc), jnp.where(rowsel, m2, m2acc))

    h1_max, h2_max = lax.fori_loop(
        0, B, _seg_max,
        (jnp.full((B, H), neg), jnp.full((B, H), neg)))

    hp = jnp.concatenate((h1_sum, h2_sum, h1_max, h2_max), axis=1)
    hh = jnp.dot(hp, Wl1_ref[...], preferred_element_type=jnp.float32) + bl1_ref[...]
    hh = jnp.maximum(hh, 0.0)
    hh = jnp.dot(hh, Wl2_ref[...], preferred_element_type=jnp.float32) + bl2_ref[...]
    lin_ref[...] = hh
    sig_ref[...] = jax.nn.sigmoid(hh)


_dense2 = pl.pallas_call(
    _dense2_body,
    out_shape=(jax.ShapeDtypeStruct((B, 1), jnp.float32),
               jax.ShapeDtypeStruct((B, 1), jnp.float32)),
)


def kernel(x, edge_index, batch, W1a, b1a, g1, be1, W1b, b1b, W2a, b2a, g2, be2,
           W2b, b2b, Wl1, bl1, Wl2, bl2):
    src = edge_index[0]
    dst = edge_index[1]
    pad = EPAD - E
    src4 = jnp.concatenate([src, jnp.zeros((pad,), jnp.int32)]).reshape(NC, NS, NCHUNK, CH)
    dst_p = jnp.concatenate([dst, jnp.full((pad,), N, jnp.int32)])
    dst4 = dst_p.reshape(NC, NS, NCHUNK, CH)

    pos, ptab = _tc_pos(dst_p.reshape(NC, EHALF))
    pos4 = pos.reshape(NC, NS, NCHUNK, CH)
    bs, bd = _make_sc_bin()(src4, dst4, pos4)
    _sc_agg = _make_sc_agg()
    agg1 = _sc_agg(x, bs, bd, ptab)
    h1 = _dense1(x, agg1, W1a, b1a.reshape(1, H), g1.reshape(1, H),
                 be1.reshape(1, H), W1b, b1b.reshape(1, H))
    agg2 = _sc_agg(h1, bs, bd, ptab)
    return _dense2(h1, agg2, batch.reshape(N, 1), W2a, b2a.reshape(1, H),
                   g2.reshape(1, H), be2.reshape(1, H), W2b, b2b.reshape(1, H),
                   Wl1, bl1.reshape(1, 4 * H), Wl2, bl2.reshape(1, 1))


# speedup vs baseline: 1.9596x; 1.0018x over previous
import functools

import jax
import jax.numpy as jnp
from jax import lax
from jax.experimental import pallas as pl
from jax.experimental.pallas import tpu as pltpu
from jax.experimental.pallas import tpu_sc as plsc

N = 10000
H = 128
B = 64
E = 320000
NC = 2
NS = 16
NW = NC * NS
CH = 128
NCHUNK = 80
EPT = NCHUNK * CH
EHALF = NS * EPT
HALFX = EHALF + CH
EPAD = NC * EHALF
TPR = 640
TPRA = 648
MDIV = 6554


def _tc_pos_body(dst_ref, pos_ref, ptab_ref):
    bins = lax.broadcasted_iota(jnp.int32, (NS, 1), 0)
    lane = lax.broadcasted_iota(jnp.int32, (NS, 128), 1)
    for c in range(NC):
        d = dst_ref[c]
        w = lax.shift_right_logical(d * MDIV, 22)
        oh = (w[None, :] == bins).astype(jnp.int32)
        incl = oh
        sh = 1
        while sh < EHALF:
            shifted = jnp.concatenate(
                [jnp.zeros((NS, sh), jnp.int32), incl[:, :EHALF - sh]], axis=1)
            incl = incl + shifted
            sh *= 2
        run = jnp.int32(0)
        starts = []
        for wb in range(NS):
            starts.append(run)
            run = run + incl[wb, EHALF - 1]
        prelv = jnp.stack(starts).reshape(NS, 1)
        pos_rel = jnp.sum(oh * (prelv + incl - 1), axis=0)
        pos_ref[c] = pos_rel
        endv = jnp.concatenate(
            [prelv[1:], jnp.full((1, 1), EHALF, jnp.int32)], axis=0)
        ptab_ref[c] = jnp.where(lane == 0, prelv,
                                jnp.where(lane == 1, endv, 0))


_tc_pos = pl.pallas_call(
    _tc_pos_body,
    out_shape=(jax.ShapeDtypeStruct((NC, EHALF), jnp.int32),
               jax.ShapeDtypeStruct((NC, NS, 128), jnp.int32)),
)


CPB = HALFX // NS


def _sc_bin_body(src_hbm, dst_hbm, pos_hbm, bs_hbm, bd_hbm,
                 sv, dv, pv, zv, bounce, bs_sp, bd_sp, qsem):
    c = lax.axis_index("c")
    s = lax.axis_index("s")

    pltpu.sync_copy(src_hbm.at[c, s], sv)
    pltpu.sync_copy(dst_hbm.at[c, s], dv)
    pltpu.sync_copy(pos_hbm.at[c, s], pv)

    @pl.when(s == 0)
    def _zp():
        for i in range(CH // 16):
            zv[pl.ds(i * 16, 16)] = jnp.zeros((16,), jnp.int32)
        pltpu.sync_copy(zv, bs_sp.at[pl.ds(EHALF, CH)])
        pltpu.sync_copy(zv, bd_sp.at[pl.ds(EHALF, CH)])

    def _grp(g, _):
        for q in range(4):
            j = g * 4 + q
            pltpu.async_copy(sv.at[j], bs_sp.at[pv.at[j]], qsem)
            pltpu.async_copy(dv.at[j], bd_sp.at[pv.at[j]], qsem)
        for q in range(4):
            j = g * 4 + q
            pltpu.make_async_copy(sv.at[j], bs_sp.at[pv.at[j]], qsem).wait()
            pltpu.make_async_copy(dv.at[j], bd_sp.at[pv.at[j]], qsem).wait()
        return 0
    lax.fori_loop(0, NCHUNK // 4, _grp, 0)
    plsc.subcore_barrier()

    hoff = pl.multiple_of(c * HALFX + s * CPB, 8)
    pltpu.sync_copy(bs_sp.at[pl.ds(s * CPB, CPB)], bounce)
    pltpu.sync_copy(bounce, bs_hbm.at[pl.ds(hoff, CPB)])
    pltpu.sync_copy(bd_sp.at[pl.ds(s * CPB, CPB)], bounce)
    pltpu.sync_copy(bounce, bd_hbm.at[pl.ds(hoff, CPB)])


@functools.lru_cache(maxsize=None)
def _make_sc_bin():
    mesh = plsc.VectorSubcoreMesh(core_axis_name="c", subcore_axis_name="s")
    return pl.kernel(
        _sc_bin_body,
        mesh=mesh,
        out_type=(jax.ShapeDtypeStruct((NC * HALFX,), jnp.int32),
                  jax.ShapeDtypeStruct((NC * HALFX,), jnp.int32)),
        scratch_types=(
            [pltpu.VMEM((NCHUNK, CH), jnp.int32)] * 3
            + [pltpu.VMEM((CH,), jnp.int32),
               pltpu.VMEM((CPB,), jnp.int32),
               pltpu.VMEM_SHARED((HALFX,), jnp.int32),
               pltpu.VMEM_SHARED((HALFX,), jnp.int32),
               pltpu.SemaphoreType.DMA]
        ),
    )


def _sc_agg_body(x_hbm, bs_hbm, bd_hbm, ptab_hbm, out_hbm,
                 pvv, si0, si1, di0, di1, bu0, bu1, acc,
                 i0, i1, g0s, g1s):
    sidx = (si0, si1)
    didx = (di0, di1)
    bufs = (bu0, bu1)
    isem = (i0, i1)
    gsem = (g0s, g1s)
    c = lax.axis_index("c")
    s = lax.axis_index("s")
    lanes = lax.iota(jnp.int32, 16)

    pltpu.sync_copy(ptab_hbm.at[c, s], pvv)
    pv_lo = pvv[pl.ds(0, 16)]
    start = pv_lo[0]
    end = pv_lo[1]
    astart = lax.shift_left(lax.shift_right_logical(start, 3), 3)
    m = lax.shift_right_logical(end - astart + (CH - 1), 7)
    fbase = pl.multiple_of(c * HALFX + astart, 8)
    rowbase = s * TPR

    def _zr(r, _):
        for g in range(H // 16):
            acc[r, pl.ds(g * 16, 16)] = jnp.zeros((16,), jnp.float32)
        return 0
    lax.fori_loop(0, TPRA, _zr, 0)

    def _fetch_idx(j, b):
        off = pl.multiple_of(fbase + j * CH, 8)
        pltpu.async_copy(bs_hbm.at[pl.ds(off, CH)], sidx[b], isem[b])
        pltpu.async_copy(bd_hbm.at[pl.ds(off, CH)], didx[b], isem[b])

    def _wait_idx(j, b):
        off = pl.multiple_of(fbase + j * CH, 8)
        pltpu.make_async_copy(bs_hbm.at[pl.ds(off, CH)], sidx[b], isem[b]).wait()
        pltpu.make_async_copy(bd_hbm.at[pl.ds(off, CH)], didx[b], isem[b]).wait()

    def _accumulate(j, b):
        gl0 = astart + j * CH
        for q in range(CH // 16):
            dvec = didx[b][pl.ds(q * 16, 16)]
            gpos = gl0 + q * 16 + lanes
            ok = jnp.logical_and(gpos >= start, gpos < end)
            ldv = jnp.where(ok, dvec - rowbase, jnp.int32(TPR))
            rows = [ldv[k] for k in range(16)]
            for k in range(16):
                e = q * 16 + k
                for g in range(H // 16):
                    plsc.addupdate(acc.at[rows[k], pl.ds(g * 16, 16)],
                                   bufs[b][e, pl.ds(g * 16, 16)])

    for b in range(2):
        @pl.when(b < m)
        def _():
            _fetch_idx(b, b)
    @pl.when(0 < m)
    def _():
        _wait_idx(0, 0)
        pltpu.async_copy(x_hbm.at[sidx[0]], bufs[0], gsem[0])

    def _outer(t, _):
        for b in range(2):
            j = t * 2 + b
            bn = (b + 1) % 2

            @pl.when(j < m)
            def _():
                pltpu.make_async_copy(x_hbm.at[sidx[b]], bufs[b], gsem[b]).wait()

                @pl.when(j + 1 < m)
                def _():
                    _wait_idx(j + 1, bn)
                    pltpu.async_copy(x_hbm.at[sidx[bn]], bufs[bn], gsem[bn])
                _accumulate(j, b)

                @pl.when(j + 2 < m)
                def _():
                    _fetch_idx(j + 2, b)
        return 0
    lax.fori_loop(0, lax.shift_right_logical(m + 1, 1), _outer, 0)

    @pl.when(s < NS - 1)
    def _cp():
        pltpu.sync_copy(acc.at[pl.ds(0, TPR)], out_hbm.at[c, pl.ds(s * TPR, TPR)])

    @pl.when(s == NS - 1)
    def _cpl():
        nlast = N - (NS - 1) * TPR
        pltpu.sync_copy(acc.at[pl.ds(0, nlast)],
                        out_hbm.at[c, pl.ds((NS - 1) * TPR, nlast)])


@functools.lru_cache(maxsize=None)
def _make_sc_agg():
    mesh = plsc.VectorSubcoreMesh(core_axis_name="c", subcore_axis_name="s")
    return pl.kernel(
        _sc_agg_body,
        mesh=mesh,
        out_type=jax.ShapeDtypeStruct((NC, N, H), jnp.float32),
        scratch_types=(
            [pltpu.VMEM((128,), jnp.int32)]
            + [pltpu.VMEM((CH,), jnp.int32)] * 4
            + [pltpu.VMEM((CH, H), jnp.float32)] * 2
            + [pltpu.VMEM((TPRA, H), jnp.float32)]
            + [pltpu.SemaphoreType.DMA] * 4
        ),
    )


def _dense1_body(x_ref, agg_ref, Wa_ref, ba_ref, g_ref, be_ref, Wb_ref, bb_ref, out_ref):
    h = x_ref[...] + agg_ref[0] + agg_ref[1]
    h = jnp.dot(h, Wa_ref[...], preferred_element_type=jnp.float32) + ba_ref[...]
    m = jnp.mean(h, axis=0, keepdims=True)
    cc = h - m
    v = jnp.mean(cc * cc, axis=0, keepdims=True)
    h = g_ref[...] * cc * lax.rsqrt(v + 1e-5) + be_ref[...]
    h = jnp.maximum(h, 0.0)
    h = jnp.dot(h, Wb_ref[...], preferred_element_type=jnp.float32) + bb_ref[...]
    out_ref[...] = jnp.maximum(h, 0.0)


_dense1 = pl.pallas_call(
    _dense1_body,
    out_shape=jax.ShapeDtypeStruct((N, H), jnp.float32),
)


def _dense2_body(h1_ref, agg_ref, batch_ref, Wa_ref, ba_ref, g_ref, be_ref,
                 Wb_ref, bb_ref, Wl1_ref, bl1_ref, Wl2_ref, bl2_ref,
                 sig_ref, lin_ref):
    h1 = h1_ref[...]
    h = h1 + agg_ref[0] + agg_ref[1]
    h = jnp.dot(h, Wa_ref[...], preferred_element_type=jnp.float32) + ba_ref[...]
    m = jnp.mean(h, axis=0, keepdims=True)
    cc = h - m
    v = jnp.mean(cc * cc, axis=0, keepdims=True)
    h = g_ref[...] * cc * lax.rsqrt(v + 1e-5) + be_ref[...]
    h = jnp.maximum(h, 0.0)
    h = jnp.dot(h, Wb_ref[...], preferred_element_type=jnp.float32) + bb_ref[...]
    h2 = jnp.maximum(h, 0.0)

    bvec = batch_ref[...]
    seg = lax.broadcasted_iota(jnp.int32, (1, B), 1)
    onehot = (bvec == seg).astype(jnp.float32)
    dn = (((0,), (0,)), ((), ()))
    h1_sum = lax.dot_general(onehot, h1, dn, preferred_element_type=jnp.float32)
    h2_sum = lax.dot_general(onehot, h2, dn, preferred_element_type=jnp.float32)

    neg = jnp.float32(-jnp.inf)
    rowid = lax.broadcasted_iota(jnp.int32, (B, 1), 0)

    def _seg_max(b, carry):
        m1acc, m2acc = carry
        mask = bvec == b
        m1 = jnp.max(jnp.where(mask, h1, neg), axis=0, keepdims=True)
        m2 = jnp.max(jnp.where(mask, h2, neg), axis=0, keepdims=True)
        rowsel = rowid == b
        return (jnp.where(rowsel, m1, m1acc), jnp.where(rowsel, m2, m2acc))

    h1_max, h2_max = lax.fori_loop(
        0, B, _seg_max,
        (jnp.full((B, H), neg), jnp.full((B, H), neg)))

    hp = jnp.concatenate((h1_sum, h2_sum, h1_max, h2_max), axis=1)
    hh = jnp.dot(hp, Wl1_ref[...], preferred_element_type=jnp.float32) + bl1_ref[...]
    hh = jnp.maximum(hh, 0.0)
    hh = jnp.dot(hh, Wl2_ref[...], preferred_element_type=jnp.float32) + bl2_ref[...]
    lin_ref[...] = hh
    sig_ref[...] = jax.nn.sigmoid(hh)


_dense2 = pl.pallas_call(
    _dense2_body,
    out_shape=(jax.ShapeDtypeStruct((B, 1), jnp.float32),
               jax.ShapeDtypeStruct((B, 1), jnp.float32)),
)


def kernel(x, edge_index, batch, W1a, b1a, g1, be1, W1b, b1b, W2a, b2a, g2, be2,
           W2b, b2b, Wl1, bl1, Wl2, bl2):
    src = edge_index[0]
    dst = edge_index[1]
    pad = EPAD - E
    src4 = jnp.concatenate([src, jnp.zeros((pad,), jnp.int32)]).reshape(NC, NS, NCHUNK, CH)
    dst_p = jnp.concatenate([dst, jnp.full((pad,), N, jnp.int32)])
    dst4 = dst_p.reshape(NC, NS, NCHUNK, CH)

    pos, ptab = _tc_pos(dst_p.reshape(NC, EHALF))
    pos4 = pos.reshape(NC, NS, NCHUNK, CH)
    bs, bd = _make_sc_bin()(src4, dst4, pos4)
    _sc_agg = _make_sc_agg()
    agg1 = _sc_agg(x, bs, bd, ptab)
    h1 = _dense1(x, agg1, W1a, b1a.reshape(1, H), g1.reshape(1, H),
                 be1.reshape(1, H), W1b, b1b.reshape(1, H))
    agg2 = _sc_agg(h1, bs, bd, ptab)
    return _dense2(h1, agg2, batch.reshape(N, 1), W2a, b2a.reshape(1, H),
                   g2.reshape(1, H), be2.reshape(1, H), W2b, b2b.reshape(1, H),
                   Wl1, bl1.reshape(1, 4 * H), Wl2, bl2.reshape(1, 1))


# speedup vs baseline: 2.1924x; 1.1188x over previous
import functools

import jax
import jax.numpy as jnp
from jax import lax
from jax.experimental import pallas as pl
from jax.experimental.pallas import tpu as pltpu
from jax.experimental.pallas import tpu_sc as plsc

N = 10000
H = 128
B = 64
E = 320000
NC = 2
NS = 16
NW = NC * NS
CH = 128
NCHUNK = 80
EPT = NCHUNK * CH
EHALF = NS * EPT
HALFX = EHALF + CH
EPAD = NC * EHALF
TPR = 640
TPRA = 648
MDIV = 6554


def _tc_pos_body(dst_ref, pos_ref, ptab_ref):
    bins = lax.broadcasted_iota(jnp.int32, (NS, 1), 0)
    lane = lax.broadcasted_iota(jnp.int32, (NS, 128), 1)
    for c in range(NC):
        d = dst_ref[c]
        w = lax.shift_right_logical(d * MDIV, 22)
        oh = (w[None, :] == bins).astype(jnp.int32)
        incl = oh
        sh = 1
        while sh < EHALF:
            shifted = jnp.concatenate(
                [jnp.zeros((NS, sh), jnp.int32), incl[:, :EHALF - sh]], axis=1)
            incl = incl + shifted
            sh *= 2
        run = jnp.int32(0)
        starts = []
        for wb in range(NS):
            starts.append(run)
            run = run + incl[wb, EHALF - 1]
        prelv = jnp.stack(starts).reshape(NS, 1)
        pos_rel = jnp.sum(oh * (prelv + incl - 1), axis=0)
        pos_ref[c] = pos_rel
        endv = jnp.concatenate(
            [prelv[1:], jnp.full((1, 1), EHALF, jnp.int32)], axis=0)
        ptab_ref[c] = jnp.where(lane == 0, prelv,
                                jnp.where(lane == 1, endv, 0))


_tc_pos = pl.pallas_call(
    _tc_pos_body,
    out_shape=(jax.ShapeDtypeStruct((NC, EHALF), jnp.int32),
               jax.ShapeDtypeStruct((NC, NS, 128), jnp.int32)),
)


CPB = HALFX // NS


def _sc_bin_body(src_hbm, dst_hbm, pos_hbm, bs_hbm, bd_hbm,
                 sv, dv, pv, zv, bounce, bs_sp, bd_sp, qsem):
    c = lax.axis_index("c")
    s = lax.axis_index("s")

    pltpu.sync_copy(src_hbm.at[c, s], sv)
    pltpu.sync_copy(dst_hbm.at[c, s], dv)
    pltpu.sync_copy(pos_hbm.at[c, s], pv)

    @pl.when(s == 0)
    def _zp():
        for i in range(CH // 16):
            zv[pl.ds(i * 16, 16)] = jnp.zeros((16,), jnp.int32)
        pltpu.sync_copy(zv, bs_sp.at[pl.ds(EHALF, CH)])
        pltpu.sync_copy(zv, bd_sp.at[pl.ds(EHALF, CH)])

    def _grp(g, _):
        for q in range(4):
            j = g * 4 + q
            pltpu.async_copy(sv.at[j], bs_sp.at[pv.at[j]], qsem)
            pltpu.async_copy(dv.at[j], bd_sp.at[pv.at[j]], qsem)
        for q in range(4):
            j = g * 4 + q
            pltpu.make_async_copy(sv.at[j], bs_sp.at[pv.at[j]], qsem).wait()
            pltpu.make_async_copy(dv.at[j], bd_sp.at[pv.at[j]], qsem).wait()
        return 0
    lax.fori_loop(0, NCHUNK // 4, _grp, 0)
    plsc.subcore_barrier()

    hoff = pl.multiple_of(c * HALFX + s * CPB, 8)
    pltpu.sync_copy(bs_sp.at[pl.ds(s * CPB, CPB)], bounce)
    pltpu.sync_copy(bounce, bs_hbm.at[pl.ds(hoff, CPB)])
    pltpu.sync_copy(bd_sp.at[pl.ds(s * CPB, CPB)], bounce)
    pltpu.sync_copy(bounce, bd_hbm.at[pl.ds(hoff, CPB)])


@functools.lru_cache(maxsize=None)
def _make_sc_bin():
    mesh = plsc.VectorSubcoreMesh(core_axis_name="c", subcore_axis_name="s")
    return pl.kernel(
        _sc_bin_body,
        mesh=mesh,
        out_type=(jax.ShapeDtypeStruct((NC * HALFX,), jnp.int32),
                  jax.ShapeDtypeStruct((NC * HALFX,), jnp.int32)),
        scratch_types=(
            [pltpu.VMEM((NCHUNK, CH), jnp.int32)] * 3
            + [pltpu.VMEM((CH,), jnp.int32),
               pltpu.VMEM((CPB,), jnp.int32),
               pltpu.VMEM_SHARED((HALFX,), jnp.int32),
               pltpu.VMEM_SHARED((HALFX,), jnp.int32),
               pltpu.SemaphoreType.DMA]
        ),
    )


def _sc_agg_body(x_hbm, bs_hbm, bd_hbm, ptab_hbm, out_hbm,
                 pvv, si0, si1, di0, di1, bu0, bu1, acc,
                 i0, i1, g0s, g1s):
    sidx = (si0, si1)
    didx = (di0, di1)
    bufs = (bu0, bu1)
    isem = (i0, i1)
    gsem = (g0s, g1s)
    c = lax.axis_index("c")
    s = lax.axis_index("s")
    lanes = lax.iota(jnp.int32, 16)

    pltpu.sync_copy(ptab_hbm.at[c, s], pvv)
    pv_lo = pvv[pl.ds(0, 16)]
    start = pv_lo[0]
    end = pv_lo[1]
    astart = lax.shift_left(lax.shift_right_logical(start, 3), 3)
    m = lax.shift_right_logical(end - astart + (CH - 1), 7)
    fbase = pl.multiple_of(c * HALFX + astart, 8)
    rowbase = s * TPR

    def _zr(r, _):
        for g in range(H // 16):
            acc[r, pl.ds(g * 16, 16)] = jnp.zeros((16,), jnp.float32)
        return 0
    lax.fori_loop(0, TPRA, _zr, 0)

    def _fetch_idx(j, b):
        off = pl.multiple_of(fbase + j * CH, 8)
        pltpu.async_copy(bs_hbm.at[pl.ds(off, CH)], sidx[b], isem[b])
        pltpu.async_copy(bd_hbm.at[pl.ds(off, CH)], didx[b], isem[b])

    def _wait_idx(j, b):
        off = pl.multiple_of(fbase + j * CH, 8)
        pltpu.make_async_copy(bs_hbm.at[pl.ds(off, CH)], sidx[b], isem[b]).wait()
        pltpu.make_async_copy(bd_hbm.at[pl.ds(off, CH)], didx[b], isem[b]).wait()

    def _accumulate(j, b):
        gl0 = astart + j * CH
        for q in range(CH // 16):
            dvec = didx[b][pl.ds(q * 16, 16)]
            gpos = gl0 + q * 16 + lanes
            ok = jnp.logical_and(gpos >= start, gpos < end)
            ldv = jnp.where(ok, dvec - rowbase, jnp.int32(TPR))
            rows = [ldv[k] for k in range(16)]
            for k in range(16):
                e = q * 16 + k
                vals = [bufs[b][e, pl.ds(g * 16, 16)] for g in range(H // 16)]
                for g in range(H // 16):
                    plsc.addupdate(acc.at[rows[k], pl.ds(g * 16, 16)], vals[g])

    for b in range(2):
        @pl.when(b < m)
        def _():
            _fetch_idx(b, b)
    @pl.when(0 < m)
    def _():
        _wait_idx(0, 0)
        pltpu.async_copy(x_hbm.at[sidx[0]], bufs[0], gsem[0])

    def _outer(t, _):
        for b in range(2):
            j = t * 2 + b
            bn = (b + 1) % 2

            @pl.when(j < m)
            def _():
                pltpu.make_async_copy(x_hbm.at[sidx[b]], bufs[b], gsem[b]).wait()

                @pl.when(j + 1 < m)
                def _():
                    _wait_idx(j + 1, bn)
                    pltpu.async_copy(x_hbm.at[sidx[bn]], bufs[bn], gsem[bn])
                _accumulate(j, b)

                @pl.when(j + 2 < m)
                def _():
                    _fetch_idx(j + 2, b)
        return 0
    lax.fori_loop(0, lax.shift_right_logical(m + 1, 1), _outer, 0)

    @pl.when(s < NS - 1)
    def _cp():
        pltpu.sync_copy(acc.at[pl.ds(0, TPR)], out_hbm.at[c, pl.ds(s * TPR, TPR)])

    @pl.when(s == NS - 1)
    def _cpl():
        nlast = N - (NS - 1) * TPR
        pltpu.sync_copy(acc.at[pl.ds(0, nlast)],
                        out_hbm.at[c, pl.ds((NS - 1) * TPR, nlast)])


@functools.lru_cache(maxsize=None)
def _make_sc_agg():
    mesh = plsc.VectorSubcoreMesh(core_axis_name="c", subcore_axis_name="s")
    return pl.kernel(
        _sc_agg_body,
        mesh=mesh,
        out_type=jax.ShapeDtypeStruct((NC, N, H), jnp.float32),
        scratch_types=(
            [pltpu.VMEM((128,), jnp.int32)]
            + [pltpu.VMEM((CH,), jnp.int32)] * 4
            + [pltpu.VMEM((CH, H), jnp.float32)] * 2
            + [pltpu.VMEM((TPRA, H), jnp.float32)]
            + [pltpu.SemaphoreType.DMA] * 4
        ),
    )


def _dense1_body(x_ref, agg_ref, Wa_ref, ba_ref, g_ref, be_ref, Wb_ref, bb_ref, out_ref):
    h = x_ref[...] + agg_ref[0] + agg_ref[1]
    h = jnp.dot(h, Wa_ref[...], preferred_element_type=jnp.float32) + ba_ref[...]
    m = jnp.mean(h, axis=0, keepdims=True)
    cc = h - m
    v = jnp.mean(cc * cc, axis=0, keepdims=True)
    h = g_ref[...] * cc * lax.rsqrt(v + 1e-5) + be_ref[...]
    h = jnp.maximum(h, 0.0)
    h = jnp.dot(h, Wb_ref[...], preferred_element_type=jnp.float32) + bb_ref[...]
    out_ref[...] = jnp.maximum(h, 0.0)


_dense1 = pl.pallas_call(
    _dense1_body,
    out_shape=jax.ShapeDtypeStruct((N, H), jnp.float32),
)


def _dense2_body(h1_ref, agg_ref, batch_ref, Wa_ref, ba_ref, g_ref, be_ref,
                 Wb_ref, bb_ref, Wl1_ref, bl1_ref, Wl2_ref, bl2_ref,
                 sig_ref, lin_ref):
    h1 = h1_ref[...]
    h = h1 + agg_ref[0] + agg_ref[1]
    h = jnp.dot(h, Wa_ref[...], preferred_element_type=jnp.float32) + ba_ref[...]
    m = jnp.mean(h, axis=0, keepdims=True)
    cc = h - m
    v = jnp.mean(cc * cc, axis=0, keepdims=True)
    h = g_ref[...] * cc * lax.rsqrt(v + 1e-5) + be_ref[...]
    h = jnp.maximum(h, 0.0)
    h = jnp.dot(h, Wb_ref[...], preferred_element_type=jnp.float32) + bb_ref[...]
    h2 = jnp.maximum(h, 0.0)

    bvec = batch_ref[...]
    seg = lax.broadcasted_iota(jnp.int32, (1, B), 1)
    onehot = (bvec == seg).astype(jnp.float32)
    dn = (((0,), (0,)), ((), ()))
    h1_sum = lax.dot_general(onehot, h1, dn, preferred_element_type=jnp.float32)
    h2_sum = lax.dot_general(onehot, h2, dn, preferred_element_type=jnp.float32)

    neg = jnp.float32(-jnp.inf)
    rowid = lax.broadcasted_iota(jnp.int32, (B, 1), 0)

    def _seg_max(b, carry):
        m1acc, m2acc = carry
        mask = bvec == b
        m1 = jnp.max(jnp.where(mask, h1, neg), axis=0, keepdims=True)
        m2 = jnp.max(jnp.where(mask, h2, neg), axis=0, keepdims=True)
        rowsel = rowid == b
        return (jnp.where(rowsel, m1, m1acc), jnp.where(rowsel, m2, m2acc))

    h1_max, h2_max = lax.fori_loop(
        0, B, _seg_max,
        (jnp.full((B, H), neg), jnp.full((B, H), neg)))

    hp = jnp.concatenate((h1_sum, h2_sum, h1_max, h2_max), axis=1)
    hh = jnp.dot(hp, Wl1_ref[...], preferred_element_type=jnp.float32) + bl1_ref[...]
    hh = jnp.maximum(hh, 0.0)
    hh = jnp.dot(hh, Wl2_ref[...], preferred_element_type=jnp.float32) + bl2_ref[...]
    lin_ref[...] = hh
    sig_ref[...] = jax.nn.sigmoid(hh)


_dense2 = pl.pallas_call(
    _dense2_body,
    out_shape=(jax.ShapeDtypeStruct((B, 1), jnp.float32),
               jax.ShapeDtypeStruct((B, 1), jnp.float32)),
)


def kernel(x, edge_index, batch, W1a, b1a, g1, be1, W1b, b1b, W2a, b2a, g2, be2,
           W2b, b2b, Wl1, bl1, Wl2, bl2):
    src = edge_index[0]
    dst = edge_index[1]
    pad = EPAD - E
    src4 = jnp.concatenate([src, jnp.zeros((pad,), jnp.int32)]).reshape(NC, NS, NCHUNK, CH)
    dst_p = jnp.concatenate([dst, jnp.full((pad,), N, jnp.int32)])
    dst4 = dst_p.reshape(NC, NS, NCHUNK, CH)

    pos, ptab = _tc_pos(dst_p.reshape(NC, EHALF))
    pos4 = pos.reshape(NC, NS, NCHUNK, CH)
    bs, bd = _make_sc_bin()(src4, dst4, pos4)
    _sc_agg = _make_sc_agg()
    agg1 = _sc_agg(x, bs, bd, ptab)
    h1 = _dense1(x, agg1, W1a, b1a.reshape(1, H), g1.reshape(1, H),
                 be1.reshape(1, H), W1b, b1b.reshape(1, H))
    agg2 = _sc_agg(h1, bs, bd, ptab)
    return _dense2(h1, agg2, batch.reshape(N, 1), W2a, b2a.reshape(1, H),
                   g2.reshape(1, H), be2.reshape(1, H), W2b, b2b.reshape(1, H),
                   Wl1, bl1.reshape(1, 4 * H), Wl2, bl2.reshape(1, 1))


# speedup vs baseline: 2.4864x; 1.1341x over previous
import functools

import jax
import jax.numpy as jnp
from jax import lax
from jax.experimental import pallas as pl
from jax.experimental.pallas import tpu as pltpu
from jax.experimental.pallas import tpu_sc as plsc

N = 10000
H = 128
B = 64
E = 320000
NC = 2
NS = 16
NW = NC * NS
CH = 128
NCHUNK = 80
EPAD = NW * NCHUNK * CH
NB = 2
NACC = 10240
ZR = 80
ROWS_T = 624
CPR = 104
TAIL0 = NS * ROWS_T
TAILR = N - TAIL0

def _sc_agg_body(x_hbm, src_hbm, dst_hbm, out_hbm, src_v, d0, d1,
                 b0, b1, acc_sh, g0, g1, s0, s1, e0, e1):
    bufs = (b0, b1)
    dring = (d0, d1)
    gsem = (g0, g1)
    ssem = (s0, s1)
    dsem = (e0, e1)
    c = lax.axis_index("c")
    s = lax.axis_index("s")
    wid = s * NC + c

    pltpu.sync_copy(src_hbm.at[wid], src_v)

    rows_v = bufs[0]

    def _zrow(i, _):
        def _zcol(j, __):
            rows_v[i, pl.ds(j * 16, 16)] = jnp.zeros((16,), jnp.float32)
            return 0
        return lax.fori_loop(0, H // 16, _zcol, 0)
    lax.fori_loop(0, ZR, _zrow, 0)
    nz = NACC // NS // ZR
    for z in range(nz):
        pltpu.sync_copy(rows_v.at[pl.ds(0, ZR)],
                        acc_sh.at[pl.ds(s * (nz * ZR) + z * ZR, ZR)])
    plsc.subcore_barrier()

    def _gather(j, b):
        pltpu.async_copy(x_hbm.at[src_v.at[j]], bufs[b], gsem[b])

    for b in range(NB):
        pltpu.async_copy(dst_hbm.at[wid, b], dring[b], dsem[b])
        _gather(b, b)

    def _outer(t, _):
        j0 = t * NB
        for b in range(NB):
            pltpu.make_async_copy(x_hbm.at[src_v.at[j0 + b]], bufs[b], gsem[b]).wait()
            pltpu.make_async_copy(dst_hbm.at[wid, j0 + b], dring[b], dsem[b]).wait()
            pltpu.async_copy(bufs[b], acc_sh.at[dring[b]], ssem[b], add=True)
        for b in range(NB):
            pltpu.make_async_copy(bufs[b], acc_sh.at[dring[b]], ssem[b]).wait()

            @pl.when(j0 + NB + b < NCHUNK)
            def _():
                pltpu.async_copy(dst_hbm.at[wid, j0 + NB + b], dring[b], dsem[b])
                _gather(j0 + NB + b, b)
        return 0
    lax.fori_loop(0, NCHUNK // NB, _outer, 0)
    plsc.subcore_barrier()

    nk = ROWS_T // CPR
    for k in range(nk):
        bk = bufs[k % 2]
        r0 = s * ROWS_T + k * CPR
        if k >= 2:
            rp = s * ROWS_T + (k - 2) * CPR
            pltpu.make_async_copy(bk.at[pl.ds(0, CPR)],
                                  out_hbm.at[c, pl.ds(rp, CPR)], ssem[k % 2]).wait()
        pltpu.sync_copy(acc_sh.at[pl.ds(r0, CPR)], bk.at[pl.ds(0, CPR)])
        pltpu.async_copy(bk.at[pl.ds(0, CPR)], out_hbm.at[c, pl.ds(r0, CPR)], ssem[k % 2])
    for k in range(nk - 2, nk):
        bk = bufs[k % 2]
        r0 = s * ROWS_T + k * CPR
        pltpu.make_async_copy(bk.at[pl.ds(0, CPR)],
                              out_hbm.at[c, pl.ds(r0, CPR)], ssem[k % 2]).wait()

    @pl.when(s == 0)
    def _tail():
        pltpu.sync_copy(acc_sh.at[pl.ds(TAIL0, TAILR)], b0.at[pl.ds(0, TAILR)])
        pltpu.sync_copy(b0.at[pl.ds(0, TAILR)], out_hbm.at[c, pl.ds(TAIL0, TAILR)])


@functools.lru_cache(maxsize=None)
def _make_sc_agg():
    mesh = plsc.VectorSubcoreMesh(core_axis_name="c", subcore_axis_name="s")
    return pl.kernel(
        _sc_agg_body,
        mesh=mesh,
        out_type=jax.ShapeDtypeStruct((NC, N, H), jnp.float32),
        scratch_types=(
            [pltpu.VMEM((NCHUNK, CH), jnp.int32)]
            + [pltpu.VMEM((CH,), jnp.int32)] * NB
            + [pltpu.VMEM((CH, H), jnp.float32)] * NB
            + [pltpu.VMEM_SHARED((NACC, H), jnp.float32)]
            + [pltpu.SemaphoreType.DMA] * (3 * NB)
        ),
    )


def _dense1_body(x_ref, agg_ref, Wa_ref, ba_ref, g_ref, be_ref, Wb_ref, bb_ref, out_ref):
    h = x_ref[...] + agg_ref[0] + agg_ref[1]
    h = jnp.dot(h, Wa_ref[...], preferred_element_type=jnp.float32) + ba_ref[...]
    m = jnp.mean(h, axis=0, keepdims=True)
    cc = h - m
    v = jnp.mean(cc * cc, axis=0, keepdims=True)
    h = g_ref[...] * cc / jnp.sqrt(v + 1e-5) + be_ref[...]
    h = jnp.maximum(h, 0.0)
    h = jnp.dot(h, Wb_ref[...], preferred_element_type=jnp.float32) + bb_ref[...]
    out_ref[...] = jnp.maximum(h, 0.0)


_dense1 = pl.pallas_call(
    _dense1_body,
    out_shape=jax.ShapeDtypeStruct((N, H), jnp.float32),
)


def _pool_head_body(h1_ref, h2_ref, batch_ref, Wl1_ref, bl1_ref, Wl2_ref, bl2_ref,
                    sig_ref, lin_ref):
    h1 = h1_ref[...]
    h2 = h2_ref[...]
    bvec = batch_ref[...]
    seg = lax.broadcasted_iota(jnp.int32, (1, B), 1)
    onehot = (bvec == seg).astype(jnp.float32)
    dn = (((0,), (0,)), ((), ()))
    h1_sum = lax.dot_general(onehot, h1, dn, preferred_element_type=jnp.float32,
                             precision=lax.Precision.HIGHEST)
    h2_sum = lax.dot_general(onehot, h2, dn, preferred_element_type=jnp.float32,
                             precision=lax.Precision.HIGHEST)

    neg = jnp.float32(-jnp.inf)
    rowid = lax.broadcasted_iota(jnp.int32, (B, 1), 0)

    def _seg_max(b, carry):
        m1acc, m2acc = carry
        mask = bvec == b
        m1 = jnp.max(jnp.where(mask, h1, neg), axis=0, keepdims=True)
        m2 = jnp.max(jnp.where(mask, h2, neg), axis=0, keepdims=True)
        rowsel = rowid == b
        return (jnp.where(rowsel, m1, m1acc), jnp.where(rowsel, m2, m2acc))

    h1_max, h2_max = lax.fori_loop(
        0, B, _seg_max,
        (jnp.full((B, H), neg), jnp.full((B, H), neg)))

    hp = jnp.concatenate((h1_sum, h2_sum, h1_max, h2_max), axis=1)
    hh = jnp.dot(hp, Wl1_ref[...], preferred_element_type=jnp.float32) + bl1_ref[...]
    hh = jnp.maximum(hh, 0.0)
    hh = jnp.dot(hh, Wl2_ref[...], preferred_element_type=jnp.float32) + bl2_ref[...]
    lin_ref[...] = hh
    sig_ref[...] = jax.nn.sigmoid(hh)


_pool_head = pl.pallas_call(
    _pool_head_body,
    out_shape=(jax.ShapeDtypeStruct((B, 1), jnp.float32),
               jax.ShapeDtypeStruct((B, 1), jnp.float32)),
)


def kernel(x, edge_index, batch, W1a, b1a, g1, be1, W1b, b1b, W2a, b2a, g2, be2,
           W2b, b2b, Wl1, bl1, Wl2, bl2):
    src = edge_index[0]
    dst = edge_index[1]
    pad = EPAD - E
    src3 = jnp.concatenate([src, jnp.zeros((pad,), jnp.int32)]).reshape(NW, NCHUNK, CH)
    dst3 = jnp.concatenate([dst, jnp.full((pad,), N, jnp.int32)]).reshape(NW, NCHUNK, CH)

    _sc_agg = _make_sc_agg()
    agg1 = _sc_agg(x, src3, dst3)
    h1 = _dense1(x, agg1, W1a, b1a.reshape(1, H), g1.reshape(1, H),
                 be1.reshape(1, H), W1b, b1b.reshape(1, H))
    agg2 = _sc_agg(h1, src3, dst3)
    h2 = _dense1(h1, agg2, W2a, b2a.reshape(1, H), g2.reshape(1, H),
                 be2.reshape(1, H), W2b, b2b.reshape(1, H))
    return _pool_head(h1, h2, batch.reshape(N, 1),
                      Wl1, bl1.reshape(1, 4 * H), Wl2, bl2.reshape(1, 1))
